# Initial kernel scaffold; baseline (speedup 1.0000x reference)
import jax
import jax.numpy as jnp
from jax.experimental import pallas as pl


def kernel(input, weight):
    raise NotImplementedError("write your pallas kernel here")


# speedup vs baseline: 2.5921x; 2.5921x over previous
import functools

import jax
import jax.numpy as jnp
from jax import lax
from jax.experimental import pallas as pl
from jax.experimental.pallas import tpu as pltpu
from jax.experimental.pallas import tpu_sc as plsc

D = 64
H = 50
B = 16384
NC = 2
NS = 16
NW = NC * NS
BPW = B // NW
BLK = 128
NBLK = BPW // BLK
LANES = 16


def _ebag_body(idx_hbm, w_hbm, out_hbm,
               idx_v, acc_v, buf0, buf1, stage_v,
               sem_a, sem0, sem1):
    cid = lax.axis_index("c")
    sid = lax.axis_index("s")
    wid = sid * NC + cid
    pltpu.sync_copy(idx_hbm.at[:, pl.ds(wid * NBLK, NBLK), :], idx_v)

    def accum(buf):
        def row(r, carry):
            for d in range(D // LANES):
                plsc.addupdate(acc_v.at[r, pl.ds(d * LANES, LANES)],
                               buf[r, pl.ds(d * LANES, LANES)])
            return carry
        lax.fori_loop(0, BLK, row, 0, unroll=4)

    def do_block(t, carry):
        def fire(j, buf, sem):
            pltpu.async_copy(w_hbm.at[idx_v.at[j, t]], buf, sem)

        def wait(buf, sem):
            pltpu.make_async_copy(w_hbm.at[idx_v.at[0, t]], buf, sem).wait()

        fire(0, acc_v, sem_a)
        fire(1, buf0, sem0)
        fire(2, buf1, sem1)
        wait(acc_v, sem_a)

        def step(k, carry):
            wait(buf0, sem0)
            accum(buf0)
            fire(2 * k + 3, buf0, sem0)
            wait(buf1, sem1)
            accum(buf1)
            fire(2 * k + 4, buf1, sem1)
            return carry
        lax.fori_loop(0, (H - 4) // 2, step, 0)

        wait(buf0, sem0)
        accum(buf0)
        fire(H - 1, buf0, sem0)
        wait(buf1, sem1)
        accum(buf1)
        wait(buf0, sem0)
        accum(buf0)

        def fin(r, carry):
            for d in range(D // LANES):
                v = acc_v[r, pl.ds(d * LANES, LANES)]
                stage_v[r, pl.ds(d * LANES, LANES)] = v * (1.0 / H)
                stage_v[r, pl.ds(D + d * LANES, LANES)] = v
            return carry
        lax.fori_loop(0, BLK, fin, 0, unroll=4)
        pltpu.sync_copy(stage_v, out_hbm.at[pl.ds(wid * BPW + t * BLK, BLK), :])
        return carry

    lax.fori_loop(0, NBLK, do_block, 0)


@jax.jit
def _ebag(idx3, weight):
    fn = functools.partial(
        pl.kernel,
        mesh=plsc.VectorSubcoreMesh(core_axis_name="c", subcore_axis_name="s"),
        out_type=jax.ShapeDtypeStruct((B, 2 * D), jnp.float32),
        compiler_params=pltpu.CompilerParams(use_tc_tiling_on_sc=False),
        scratch_types=[
            pltpu.VMEM((H, NBLK, BLK), jnp.int32),
            pltpu.VMEM((BLK, D), jnp.float32),
            pltpu.VMEM((BLK, D), jnp.float32),
            pltpu.VMEM((BLK, D), jnp.float32),
            pltpu.VMEM((BLK, 2 * D), jnp.float32),
            pltpu.SemaphoreType.DMA,
            pltpu.SemaphoreType.DMA,
            pltpu.SemaphoreType.DMA,
        ],
    )(_ebag_body)
    return fn(idx3, weight)


def kernel(input, weight):
    idx3 = jnp.transpose(input, (1, 0)).reshape(H, B // BLK, BLK)
    return _ebag(idx3.astype(jnp.int32), weight)
